# Initial kernel scaffold; baseline (speedup 1.0000x reference)
import jax
import jax.numpy as jnp
from jax.experimental import pallas as pl


def kernel(x, edge_index_coarse, edge_index_fine, pos_fine, mask, W1m, W1s, b1, W2m, W2s, b2, Wsm, Wss, bs, gamma, beta):
    raise NotImplementedError("write your pallas kernel here")


# speedup vs baseline: 3.4166x; 3.4166x over previous
import functools

import jax
import jax.numpy as jnp
from jax import lax
from jax.experimental import pallas as pl
from jax.experimental.pallas import tpu as pltpu
from jax.experimental.pallas import tpu_sc as plsc

N_COARSE = 5000
N_FINE = 10000
C_IN = 128
C_MID = 64
C_OUT = 128
K_KNN = 3
D_CAT = C_IN + C_MID
D_Z = C_OUT + C_MID

NC_SC = 2
NS_SC = 16
NW = NC_SC * NS_SC
CH = 128

_NEG_SLOPE = 0.01


def _leaky(v):
    return jnp.where(v > 0, v, _NEG_SLOPE * v)


def _ceil_to(x, m):
    return (x + m - 1) // m * m


def _make_segsum(n_src, d, e_pad, r_out):
    e_tile = e_pad // NW
    nch = e_tile // CH
    rpt = r_out // NS_SC
    mesh = plsc.VectorSubcoreMesh(core_axis_name="c", subcore_axis_name="s")

    @functools.partial(
        pl.kernel,
        out_type=jax.ShapeDtypeStruct((NC_SC, r_out, d), jnp.float32),
        mesh=mesh,
        scratch_types=[
            pltpu.VMEM((CH,), jnp.int32),
            pltpu.VMEM((CH,), jnp.int32),
            pltpu.VMEM((CH, d), jnp.float32),
            pltpu.VMEM_SHARED((r_out, d), jnp.float32),
            pltpu.SemaphoreType.DMA,
        ],
        compiler_params=pltpu.CompilerParams(use_tc_tiling_on_sc=False),
    )
    def segsum(feat_hbm, src_hbm, dst_hbm, zero_hbm, out_hbm,
               idx_src, idx_dst, rows, acc, sem):
        cid = lax.axis_index("c")
        sid = lax.axis_index("s")
        wid = cid * NS_SC + sid
        pltpu.sync_copy(zero_hbm, acc.at[pl.ds(sid * rpt, rpt)])
        plsc.subcore_barrier()

        def body(c, carry):
            base = wid * e_tile + c * CH
            pltpu.sync_copy(src_hbm.at[pl.ds(base, CH)], idx_src)
            pltpu.sync_copy(dst_hbm.at[pl.ds(base, CH)], idx_dst)
            pltpu.async_copy(feat_hbm.at[idx_src], rows, sem).wait()
            pltpu.sync_copy(rows, acc.at[idx_dst], add=True)
            return carry

        lax.fori_loop(0, nch, body, 0)
        plsc.subcore_barrier()
        pltpu.sync_copy(acc.at[pl.ds(sid * rpt, rpt)],
                        out_hbm.at[cid, pl.ds(sid * rpt, rpt)])

    return segsum


def _make_gather(n_src, d, b_pad):
    b_tile = b_pad // NW
    nch = b_tile // CH
    mesh = plsc.VectorSubcoreMesh(core_axis_name="c", subcore_axis_name="s")

    @functools.partial(
        pl.kernel,
        out_type=jax.ShapeDtypeStruct((b_pad, d), jnp.float32),
        mesh=mesh,
        scratch_types=[
            pltpu.VMEM((CH,), jnp.int32),
            pltpu.VMEM((CH, d), jnp.float32),
            pltpu.SemaphoreType.DMA,
        ],
        compiler_params=pltpu.CompilerParams(use_tc_tiling_on_sc=False),
    )
    def gather(tab_hbm, idx_hbm, out_hbm, idx_v, rows, sem):
        cid = lax.axis_index("c")
        sid = lax.axis_index("s")
        wid = cid * NS_SC + sid

        def body(c, carry):
            base = wid * b_tile + c * CH
            pltpu.sync_copy(idx_hbm.at[pl.ds(base, CH)], idx_v)
            pltpu.async_copy(tab_hbm.at[idx_v], rows, sem).wait()
            pltpu.sync_copy(rows, out_hbm.at[pl.ds(base, CH)])
            return carry

        lax.fori_loop(0, nch, body, 0)

    return gather


def _mm_kernel(x_ref, w_ref, o_ref):
    o_ref[...] = jnp.dot(x_ref[...], w_ref[...],
                         preferred_element_type=jnp.float32)


def _matmul(x, w, bm):
    m, k = x.shape
    n = w.shape[1]
    return pl.pallas_call(
        _mm_kernel,
        grid=(m // bm,),
        in_specs=[
            pl.BlockSpec((bm, k), lambda i: (i, 0)),
            pl.BlockSpec((k, n), lambda i: (0, 0)),
        ],
        out_specs=pl.BlockSpec((bm, n), lambda i: (i, 0)),
        out_shape=jax.ShapeDtypeStruct((m, n), jnp.float32),
    )(x, w)


def _x1cat_kernel(p_ref, y1s_ref, b1_ref, x_ref, o_ref):
    agg = p_ref[0] + p_ref[1]
    x1 = _leaky(agg + y1s_ref[...] + b1_ref[...])
    o_ref[...] = jnp.concatenate([x_ref[...], x1], axis=1)


def _knn_kernel(pf_ref, pc_ref, idx_ref, wn_ref, *, bf, ncp):
    pf = pf_ref[...]
    pc = pc_ref[...]
    d2 = (jnp.sum(pf * pf, axis=1)[:, None]
          + jnp.sum(pc * pc, axis=1)[None, :]
          - 2.0 * jnp.dot(pf, pc.T, preferred_element_type=jnp.float32))
    col = lax.broadcasted_iota(jnp.int32, (bf, ncp), 1)
    ws = []
    ids = []
    for _ in range(K_KNN):
        m = jnp.min(d2, axis=1)
        am = jnp.min(jnp.where(d2 == m[:, None], col, jnp.int32(2**30)),
                     axis=1)
        ws.append(1.0 / jnp.clip(m, 1e-16, None))
        ids.append(am)
        d2 = jnp.where(col == am[:, None], jnp.float32(jnp.inf), d2)
    den = ws[0] + ws[1] + ws[2]
    zi = jnp.zeros((bf,), jnp.int32)
    zf = jnp.zeros((bf,), jnp.float32)
    idx_ref[...] = jnp.stack(ids + [zi] * (8 - K_KNN), axis=1)
    wn_ref[...] = jnp.stack([w / den for w in ws] + [zf] * (8 - K_KNN),
                            axis=1)


def _mix_kernel(g0_ref, g1_ref, g2_ref, wn_ref, wsm_ref, wss_ref, w2s_ref,
                bs_ref, b2_ref, z1_ref, z2_ref, ps_ref, p2_ref):
    wn = wn_ref[...]
    up = (wn[:, 0][:, None] * g0_ref[...]
          + wn[:, 1][:, None] * g1_ref[...]
          + wn[:, 2][:, None] * g2_ref[...])
    x_up0 = up[:, :C_IN]
    x1_up = up[:, C_IN:]
    z1_ref[...] = jnp.dot(x_up0, wsm_ref[...],
                          preferred_element_type=jnp.float32)
    z2_ref[...] = x1_up
    ps_ref[...] = (jnp.dot(x_up0, wss_ref[...],
                           preferred_element_type=jnp.float32) + bs_ref[...])
    p2_ref[...] = (jnp.dot(x1_up, w2s_ref[...],
                           preferred_element_type=jnp.float32) + b2_ref[...])


def _final_kernel(p1_ref, p2s_ref, ps_ref, p2_ref, w2m_ref, gamma_ref,
                  beta_ref, o_ref):
    agg_skip = p1_ref[0] + p1_ref[1]
    agg2 = jnp.dot(p2s_ref[0] + p2s_ref[1], w2m_ref[...],
                   preferred_element_type=jnp.float32)
    x_skip = _leaky(agg_skip + ps_ref[...])
    x2 = _leaky(agg2 + p2_ref[...])
    y = x2 + x_skip
    mu = jnp.mean(y, axis=0)
    var = jnp.mean((y - mu[None, :]) ** 2, axis=0)
    bn = (y - mu[None, :]) / jnp.sqrt(var + 1e-5)[None, :] * gamma_ref[...] \
        + beta_ref[...]
    o_ref[...] = _leaky(bn)


def kernel(x, edge_index_coarse, edge_index_fine, pos_fine, mask,
           W1m, W1s, b1, W2m, W2s, b2, Wsm, Wss, bs, gamma, beta):
    f32 = jnp.float32
    e_c = edge_index_coarse.shape[1]
    e_f = edge_index_fine.shape[1]

    ec_pad = _ceil_to(e_c, NW * CH)
    ef_pad = _ceil_to(e_f, NW * CH)
    rc = _ceil_to(N_COARSE + 1, NS_SC * 8)
    rf = _ceil_to(N_FINE + 1, NS_SC * 8)
    b_int = _ceil_to(K_KNN * N_FINE, NW * CH)

    def pad_edges(ei, e, e_pad, dummy):
        src = jnp.concatenate(
            [ei[0].astype(jnp.int32), jnp.zeros((e_pad - e,), jnp.int32)])
        dst = jnp.concatenate(
            [ei[1].astype(jnp.int32),
             jnp.full((e_pad - e,), dummy, jnp.int32)])
        return src, dst

    src_c, dst_c = pad_edges(edge_index_coarse, e_c, ec_pad, N_COARSE)
    src_f, dst_f = pad_edges(edge_index_fine, e_f, ef_pad, N_FINE)

    zero_c = jnp.zeros((rc // NS_SC, C_MID), f32)
    zero_f1 = jnp.zeros((rf // NS_SC, C_OUT), f32)
    zero_f2 = jnp.zeros((rf // NS_SC, C_MID), f32)

    w1 = jnp.concatenate([W1m, W1s], axis=1)
    y1 = _matmul(x, w1, bm=1000)
    y1m = y1[:, :C_MID]
    y1s = y1[:, C_MID:]

    segsum_c = _make_segsum(N_COARSE, C_MID, ec_pad, rc)
    pc_part = segsum_c(y1m, src_c, dst_c, zero_c)
    pc_real = pc_part[:, :N_COARSE, :]

    xcat = pl.pallas_call(
        _x1cat_kernel,
        grid=(5,),
        in_specs=[
            pl.BlockSpec((NC_SC, 1000, C_MID), lambda i: (0, i, 0)),
            pl.BlockSpec((1000, C_MID), lambda i: (i, 0)),
            pl.BlockSpec((1, C_MID), lambda i: (0, 0)),
            pl.BlockSpec((1000, C_IN), lambda i: (i, 0)),
        ],
        out_specs=pl.BlockSpec((1000, D_CAT), lambda i: (i, 0)),
        out_shape=jax.ShapeDtypeStruct((N_COARSE, D_CAT), f32),
    )(pc_real, y1s, b1.reshape(1, C_MID), x)

    ncp = _ceil_to(N_COARSE, 8)
    pos_c = pos_fine[mask]
    pos_cp = jnp.concatenate(
        [pos_c, jnp.full((ncp - N_COARSE, 3), 1e6, f32)])
    bf = 1000
    idx8, wn8 = pl.pallas_call(
        functools.partial(_knn_kernel, bf=bf, ncp=ncp),
        grid=(N_FINE // bf,),
        in_specs=[
            pl.BlockSpec((bf, 3), lambda i: (i, 0)),
            pl.BlockSpec((ncp, 3), lambda i: (0, 0)),
        ],
        out_specs=[
            pl.BlockSpec((bf, 8), lambda i: (i, 0)),
            pl.BlockSpec((bf, 8), lambda i: (i, 0)),
        ],
        out_shape=[
            jax.ShapeDtypeStruct((N_FINE, 8), jnp.int32),
            jax.ShapeDtypeStruct((N_FINE, 8), f32),
        ],
    )(pos_fine, pos_cp)

    idx_all = jnp.concatenate(
        [idx8[:, 0], idx8[:, 1], idx8[:, 2],
         jnp.zeros((b_int - K_KNN * N_FINE,), jnp.int32)])
    gather = _make_gather(N_COARSE, D_CAT, b_int)
    g = gather(xcat, idx_all)
    g0 = g[:N_FINE]
    g1 = g[N_FINE:2 * N_FINE]
    g2 = g[2 * N_FINE:3 * N_FINE]

    z1, z2, pre_s, pre_2 = pl.pallas_call(
        _mix_kernel,
        grid=(N_FINE // bf,),
        in_specs=[
            pl.BlockSpec((bf, D_CAT), lambda i: (i, 0)),
            pl.BlockSpec((bf, D_CAT), lambda i: (i, 0)),
            pl.BlockSpec((bf, D_CAT), lambda i: (i, 0)),
            pl.BlockSpec((bf, 8), lambda i: (i, 0)),
            pl.BlockSpec((C_IN, C_OUT), lambda i: (0, 0)),
            pl.BlockSpec((C_IN, C_OUT), lambda i: (0, 0)),
            pl.BlockSpec((C_MID, C_OUT), lambda i: (0, 0)),
            pl.BlockSpec((1, C_OUT), lambda i: (0, 0)),
            pl.BlockSpec((1, C_OUT), lambda i: (0, 0)),
        ],
        out_specs=[
            pl.BlockSpec((bf, C_OUT), lambda i: (i, 0)),
            pl.BlockSpec((bf, C_MID), lambda i: (i, 0)),
            pl.BlockSpec((bf, C_OUT), lambda i: (i, 0)),
            pl.BlockSpec((bf, C_OUT), lambda i: (i, 0)),
        ],
        out_shape=[
            jax.ShapeDtypeStruct((N_FINE, C_OUT), f32),
            jax.ShapeDtypeStruct((N_FINE, C_MID), f32),
            jax.ShapeDtypeStruct((N_FINE, C_OUT), f32),
            jax.ShapeDtypeStruct((N_FINE, C_OUT), f32),
        ],
    )(g0, g1, g2, wn8, Wsm, Wss, W2s,
      bs.reshape(1, C_OUT), b2.reshape(1, C_OUT))

    segsum_f1 = _make_segsum(N_FINE, C_OUT, ef_pad, rf)
    pf1 = segsum_f1(z1, src_f, dst_f, zero_f1)[:, :N_FINE, :]
    segsum_f2 = _make_segsum(N_FINE, C_MID, ef_pad, rf)
    pf2 = segsum_f2(z2, src_f, dst_f, zero_f2)[:, :N_FINE, :]

    out = pl.pallas_call(
        _final_kernel,
        in_specs=[
            pl.BlockSpec((NC_SC, N_FINE, C_OUT), lambda: (0, 0, 0)),
            pl.BlockSpec((NC_SC, N_FINE, C_MID), lambda: (0, 0, 0)),
            pl.BlockSpec((N_FINE, C_OUT), lambda: (0, 0)),
            pl.BlockSpec((N_FINE, C_OUT), lambda: (0, 0)),
            pl.BlockSpec((C_MID, C_OUT), lambda: (0, 0)),
            pl.BlockSpec((1, C_OUT), lambda: (0, 0)),
            pl.BlockSpec((1, C_OUT), lambda: (0, 0)),
        ],
        out_specs=pl.BlockSpec((N_FINE, C_OUT), lambda: (0, 0)),
        out_shape=jax.ShapeDtypeStruct((N_FINE, C_OUT), f32),
    )(pf1, pf2, pre_s, pre_2, W2m,
      gamma.reshape(1, C_OUT), beta.reshape(1, C_OUT))
    return out


# speedup vs baseline: 5.5936x; 1.6372x over previous
import functools

import jax
import jax.numpy as jnp
from jax import lax
from jax.experimental import pallas as pl
from jax.experimental.pallas import tpu as pltpu
from jax.experimental.pallas import tpu_sc as plsc

N_COARSE = 5000
N_FINE = 10000
C_IN = 128
C_MID = 64
C_OUT = 128
K_KNN = 3
D_CAT = C_IN + C_MID
D_Z = C_OUT + C_MID

NC_SC = 2
NS_SC = 16
NW = NC_SC * NS_SC
CH = 100

_NEG_SLOPE = 0.01


def _leaky(v):
    return jnp.where(v > 0, v, _NEG_SLOPE * v)


def _ceil_to(x, m):
    return (x + m - 1) // m * m


NBUF = 2


def _make_segsum(n_src, d, e_pad, r_out):
    e_tile = e_pad // NW
    nch = e_tile // CH
    assert nch % NBUF == 0
    rpt = r_out // NS_SC
    mesh = plsc.VectorSubcoreMesh(core_axis_name="c", subcore_axis_name="s")

    @functools.partial(
        pl.kernel,
        out_type=jax.ShapeDtypeStruct((NC_SC, r_out, d), jnp.float32),
        mesh=mesh,
        scratch_types=[
            pltpu.VMEM((nch, CH), jnp.int32),
            pltpu.VMEM((nch, CH), jnp.int32),
            [pltpu.VMEM((CH, d), jnp.float32) for _ in range(NBUF)],
            pltpu.VMEM_SHARED((r_out, d), jnp.float32),
            pltpu.SemaphoreType.DMA,
        ],
        compiler_params=pltpu.CompilerParams(use_tc_tiling_on_sc=False),
    )
    def segsum(feat_hbm, src_hbm, dst_hbm, zero_hbm, out_hbm,
               src_v, dst_v, rows, acc, sem):
        cid = lax.axis_index("c")
        sid = lax.axis_index("s")
        wid = cid * NS_SC + sid
        pltpu.sync_copy(src_hbm.at[wid], src_v)
        pltpu.sync_copy(dst_hbm.at[wid], dst_v)
        pltpu.sync_copy(zero_hbm, acc.at[pl.ds(sid * rpt, rpt)])
        plsc.subcore_barrier()

        def body(i, carry):
            c = i * NBUF
            descs = [
                pltpu.async_copy(feat_hbm.at[src_v.at[c + b]], rows[b], sem)
                for b in range(NBUF)
            ]
            for b in range(NBUF):
                descs[b].wait()
                pltpu.sync_copy(rows[b], acc.at[dst_v.at[c + b]], add=True)
            return carry

        lax.fori_loop(0, nch // NBUF, body, 0)
        plsc.subcore_barrier()
        pltpu.sync_copy(acc.at[pl.ds(sid * rpt, rpt)],
                        out_hbm.at[cid, pl.ds(sid * rpt, rpt)])

    return segsum


def _make_gather(n_src, d, b_pad):
    b_tile = b_pad // NW
    nch = b_tile // CH
    assert nch % NBUF == 0
    mesh = plsc.VectorSubcoreMesh(core_axis_name="c", subcore_axis_name="s")

    @functools.partial(
        pl.kernel,
        out_type=jax.ShapeDtypeStruct((b_pad, d), jnp.float32),
        mesh=mesh,
        scratch_types=[
            pltpu.VMEM((nch, CH), jnp.int32),
            [pltpu.VMEM((CH, d), jnp.float32) for _ in range(NBUF)],
            pltpu.SemaphoreType.DMA,
        ],
        compiler_params=pltpu.CompilerParams(use_tc_tiling_on_sc=False),
    )
    def gather(tab_hbm, idx_hbm, out_hbm, idx_v, rows, sem):
        cid = lax.axis_index("c")
        sid = lax.axis_index("s")
        wid = cid * NS_SC + sid
        pltpu.sync_copy(idx_hbm.at[wid], idx_v)

        def body(i, carry):
            c = i * NBUF
            descs = [
                pltpu.async_copy(tab_hbm.at[idx_v.at[c + b]], rows[b], sem)
                for b in range(NBUF)
            ]
            for b in range(NBUF):
                descs[b].wait()
                base = wid * b_tile + (c + b) * CH
                pltpu.sync_copy(rows[b], out_hbm.at[pl.ds(base, CH)])
            return carry

        lax.fori_loop(0, nch // NBUF, body, 0)

    return gather


def _mm_kernel(x_ref, w_ref, o_ref):
    o_ref[...] = jnp.dot(x_ref[...], w_ref[...],
                         preferred_element_type=jnp.float32)


def _matmul(x, w, bm):
    m, k = x.shape
    n = w.shape[1]
    return pl.pallas_call(
        _mm_kernel,
        grid=(m // bm,),
        in_specs=[
            pl.BlockSpec((bm, k), lambda i: (i, 0)),
            pl.BlockSpec((k, n), lambda i: (0, 0)),
        ],
        out_specs=pl.BlockSpec((bm, n), lambda i: (i, 0)),
        out_shape=jax.ShapeDtypeStruct((m, n), jnp.float32),
    )(x, w)


def _x1cat_kernel(p_ref, y1s_ref, b1_ref, x_ref, o_ref):
    agg = p_ref[0] + p_ref[1]
    x1 = _leaky(agg + y1s_ref[...] + b1_ref[...])
    o_ref[...] = jnp.concatenate([x_ref[...], x1], axis=1)


def _knn_kernel(pf_ref, pc_ref, idx_ref, wn_ref, *, bf, ncp):
    pf = pf_ref[...]
    pc = pc_ref[...]
    d2 = (jnp.sum(pf * pf, axis=1)[:, None]
          + jnp.sum(pc * pc, axis=1)[None, :]
          - 2.0 * jnp.dot(pf, pc.T, preferred_element_type=jnp.float32))
    col = lax.broadcasted_iota(jnp.int32, (bf, ncp), 1)
    ws = []
    ids = []
    for _ in range(K_KNN):
        m = jnp.min(d2, axis=1)
        am = jnp.min(jnp.where(d2 == m[:, None], col, jnp.int32(2**30)),
                     axis=1)
        ws.append(1.0 / jnp.clip(m, 1e-16, None))
        ids.append(am)
        d2 = jnp.where(col == am[:, None], jnp.float32(jnp.inf), d2)
    den = ws[0] + ws[1] + ws[2]
    zi = jnp.zeros((bf,), jnp.int32)
    zf = jnp.zeros((bf,), jnp.float32)
    idx_ref[...] = jnp.stack(ids + [zi] * (8 - K_KNN), axis=1)
    wn_ref[...] = jnp.stack([w / den for w in ws] + [zf] * (8 - K_KNN),
                            axis=1)


def _mix_kernel(g0_ref, g1_ref, g2_ref, wn_ref, wsm_ref, wss_ref, w2s_ref,
                bs_ref, b2_ref, z1_ref, z2_ref, ps_ref, p2_ref):
    wn = wn_ref[...]
    up = (wn[:, 0][:, None] * g0_ref[...]
          + wn[:, 1][:, None] * g1_ref[...]
          + wn[:, 2][:, None] * g2_ref[...])
    x_up0 = up[:, :C_IN]
    x1_up = up[:, C_IN:]
    z1_ref[...] = jnp.dot(x_up0, wsm_ref[...],
                          preferred_element_type=jnp.float32)
    z2_ref[...] = x1_up
    ps_ref[...] = (jnp.dot(x_up0, wss_ref[...],
                           preferred_element_type=jnp.float32) + bs_ref[...])
    p2_ref[...] = (jnp.dot(x1_up, w2s_ref[...],
                           preferred_element_type=jnp.float32) + b2_ref[...])


def _final_kernel(p1_ref, p2s_ref, ps_ref, p2_ref, w2m_ref, gamma_ref,
                  beta_ref, o_ref):
    agg_skip = p1_ref[0] + p1_ref[1]
    agg2 = jnp.dot(p2s_ref[0] + p2s_ref[1], w2m_ref[...],
                   preferred_element_type=jnp.float32)
    x_skip = _leaky(agg_skip + ps_ref[...])
    x2 = _leaky(agg2 + p2_ref[...])
    y = x2 + x_skip
    mu = jnp.mean(y, axis=0)
    var = jnp.mean((y - mu[None, :]) ** 2, axis=0)
    bn = (y - mu[None, :]) / jnp.sqrt(var + 1e-5)[None, :] * gamma_ref[...] \
        + beta_ref[...]
    o_ref[...] = _leaky(bn)


def kernel(x, edge_index_coarse, edge_index_fine, pos_fine, mask,
           W1m, W1s, b1, W2m, W2s, b2, Wsm, Wss, bs, gamma, beta):
    f32 = jnp.float32
    e_c = edge_index_coarse.shape[1]
    e_f = edge_index_fine.shape[1]

    ec_pad = _ceil_to(e_c, NW * CH * NBUF)
    ef_pad = _ceil_to(e_f, NW * CH * NBUF)
    rc = _ceil_to(N_COARSE + 1, NS_SC * 8)
    rf = _ceil_to(N_FINE + 1, NS_SC * 8)
    b_int = _ceil_to(K_KNN * N_FINE, NW * CH * NBUF)

    def pad_edges(ei, e, e_pad, dummy):
        src = jnp.concatenate(
            [ei[0].astype(jnp.int32), jnp.zeros((e_pad - e,), jnp.int32)])
        dst = jnp.concatenate(
            [ei[1].astype(jnp.int32),
             jnp.full((e_pad - e,), dummy, jnp.int32)])
        nch = e_pad // (NW * CH)
        return src.reshape(NW, nch, CH), dst.reshape(NW, nch, CH)

    src_c, dst_c = pad_edges(edge_index_coarse, e_c, ec_pad, N_COARSE)
    src_f, dst_f = pad_edges(edge_index_fine, e_f, ef_pad, N_FINE)

    zero_c = jnp.zeros((rc // NS_SC, C_MID), f32)
    zero_f1 = jnp.zeros((rf // NS_SC, C_OUT), f32)
    zero_f2 = jnp.zeros((rf // NS_SC, C_MID), f32)

    w1 = jnp.concatenate([W1m, W1s], axis=1)
    y1 = _matmul(x, w1, bm=1000)
    y1m = y1[:, :C_MID]
    y1s = y1[:, C_MID:]

    segsum_c = _make_segsum(N_COARSE, C_MID, ec_pad, rc)
    pc_part = segsum_c(y1m, src_c, dst_c, zero_c)
    pc_real = pc_part[:, :N_COARSE, :]

    xcat = pl.pallas_call(
        _x1cat_kernel,
        grid=(5,),
        in_specs=[
            pl.BlockSpec((NC_SC, 1000, C_MID), lambda i: (0, i, 0)),
            pl.BlockSpec((1000, C_MID), lambda i: (i, 0)),
            pl.BlockSpec((1, C_MID), lambda i: (0, 0)),
            pl.BlockSpec((1000, C_IN), lambda i: (i, 0)),
        ],
        out_specs=pl.BlockSpec((1000, D_CAT), lambda i: (i, 0)),
        out_shape=jax.ShapeDtypeStruct((N_COARSE, D_CAT), f32),
    )(pc_real, y1s, b1.reshape(1, C_MID), x)

    ncp = _ceil_to(N_COARSE, 8)
    pos_c = pos_fine[mask]
    pos_cp = jnp.concatenate(
        [pos_c, jnp.full((ncp - N_COARSE, 3), 1e6, f32)])
    bf = 1000
    idx8, wn8 = pl.pallas_call(
        functools.partial(_knn_kernel, bf=bf, ncp=ncp),
        grid=(N_FINE // bf,),
        in_specs=[
            pl.BlockSpec((bf, 3), lambda i: (i, 0)),
            pl.BlockSpec((ncp, 3), lambda i: (0, 0)),
        ],
        out_specs=[
            pl.BlockSpec((bf, 8), lambda i: (i, 0)),
            pl.BlockSpec((bf, 8), lambda i: (i, 0)),
        ],
        out_shape=[
            jax.ShapeDtypeStruct((N_FINE, 8), jnp.int32),
            jax.ShapeDtypeStruct((N_FINE, 8), f32),
        ],
    )(pos_fine, pos_cp)

    idx_all = jnp.concatenate(
        [idx8[:, 0], idx8[:, 1], idx8[:, 2],
         jnp.zeros((b_int - K_KNN * N_FINE,), jnp.int32)]
    ).reshape(NW, b_int // (NW * CH), CH)
    gather = _make_gather(N_COARSE, D_CAT, b_int)
    g = gather(xcat, idx_all)
    g0 = g[:N_FINE]
    g1 = g[N_FINE:2 * N_FINE]
    g2 = g[2 * N_FINE:3 * N_FINE]

    z1, z2, pre_s, pre_2 = pl.pallas_call(
        _mix_kernel,
        grid=(N_FINE // bf,),
        in_specs=[
            pl.BlockSpec((bf, D_CAT), lambda i: (i, 0)),
            pl.BlockSpec((bf, D_CAT), lambda i: (i, 0)),
            pl.BlockSpec((bf, D_CAT), lambda i: (i, 0)),
            pl.BlockSpec((bf, 8), lambda i: (i, 0)),
            pl.BlockSpec((C_IN, C_OUT), lambda i: (0, 0)),
            pl.BlockSpec((C_IN, C_OUT), lambda i: (0, 0)),
            pl.BlockSpec((C_MID, C_OUT), lambda i: (0, 0)),
            pl.BlockSpec((1, C_OUT), lambda i: (0, 0)),
            pl.BlockSpec((1, C_OUT), lambda i: (0, 0)),
        ],
        out_specs=[
            pl.BlockSpec((bf, C_OUT), lambda i: (i, 0)),
            pl.BlockSpec((bf, C_MID), lambda i: (i, 0)),
            pl.BlockSpec((bf, C_OUT), lambda i: (i, 0)),
            pl.BlockSpec((bf, C_OUT), lambda i: (i, 0)),
        ],
        out_shape=[
            jax.ShapeDtypeStruct((N_FINE, C_OUT), f32),
            jax.ShapeDtypeStruct((N_FINE, C_MID), f32),
            jax.ShapeDtypeStruct((N_FINE, C_OUT), f32),
            jax.ShapeDtypeStruct((N_FINE, C_OUT), f32),
        ],
    )(g0, g1, g2, wn8, Wsm, Wss, W2s,
      bs.reshape(1, C_OUT), b2.reshape(1, C_OUT))

    segsum_f1 = _make_segsum(N_FINE, C_OUT, ef_pad, rf)
    pf1 = segsum_f1(z1, src_f, dst_f, zero_f1)[:, :N_FINE, :]
    segsum_f2 = _make_segsum(N_FINE, C_MID, ef_pad, rf)
    pf2 = segsum_f2(z2, src_f, dst_f, zero_f2)[:, :N_FINE, :]

    out = pl.pallas_call(
        _final_kernel,
        in_specs=[
            pl.BlockSpec((NC_SC, N_FINE, C_OUT), lambda: (0, 0, 0)),
            pl.BlockSpec((NC_SC, N_FINE, C_MID), lambda: (0, 0, 0)),
            pl.BlockSpec((N_FINE, C_OUT), lambda: (0, 0)),
            pl.BlockSpec((N_FINE, C_OUT), lambda: (0, 0)),
            pl.BlockSpec((C_MID, C_OUT), lambda: (0, 0)),
            pl.BlockSpec((1, C_OUT), lambda: (0, 0)),
            pl.BlockSpec((1, C_OUT), lambda: (0, 0)),
        ],
        out_specs=pl.BlockSpec((N_FINE, C_OUT), lambda: (0, 0)),
        out_shape=jax.ShapeDtypeStruct((N_FINE, C_OUT), f32),
    )(pf1, pf2, pre_s, pre_2, W2m,
      gamma.reshape(1, C_OUT), beta.reshape(1, C_OUT))
    return out
